# Initial kernel scaffold; baseline (speedup 1.0000x reference)
import jax
import jax.numpy as jnp
from jax.experimental import pallas as pl


def kernel(x, edge_index, edge_attr, Wn, bn, We, be, Wg, bg):
    raise NotImplementedError("write your pallas kernel here")


# speedup vs baseline: 2.7003x; 2.7003x over previous
import functools

import jax
import jax.numpy as jnp
from jax import lax
from jax.experimental import pallas as pl
from jax.experimental.pallas import tpu as pltpu
from jax.experimental.pallas import tpu_sc as plsc

N_NODES = 10000
N_EDGES = 320000
D = 128
DE = 16

NC = 2
NS = 16
NW = NC * NS
E_W = N_EDGES // NW
CH = 80
NCHUNK = E_W // CH
RPT = 624


def _leaky(v):
    return jnp.maximum(v, 0.01 * v)


def _node_dense(x, Wn, bn, Wgn):
    def body(x_ref, wn_ref, bn_ref, wgn_ref, o_ref):
        h = jnp.dot(x_ref[...], wn_ref[...], preferred_element_type=jnp.float32)
        h = _leaky(h + bn_ref[...])
        o_ref[...] = jnp.dot(h, wgn_ref[...], preferred_element_type=jnp.float32)

    return pl.pallas_call(
        body,
        out_shape=jax.ShapeDtypeStruct((N_NODES, D), jnp.float32),
    )(x, Wn, bn, Wgn)


def _edge_dense(ea, We, be, Wge, bg):
    BLK = 2560
    grid = N_EDGES // BLK

    def body(a_ref, we_ref, be_ref, wge_ref, bg_ref, o_ref):
        h = _leaky(jnp.dot(a_ref[...], we_ref[...], preferred_element_type=jnp.float32) + be_ref[...])
        o_ref[...] = jnp.dot(h, wge_ref[...], preferred_element_type=jnp.float32) + bg_ref[...]

    return pl.pallas_call(
        body,
        grid=(grid,),
        in_specs=[
            pl.BlockSpec((BLK, DE), lambda i: (i, 0)),
            pl.BlockSpec((DE, DE), lambda i: (0, 0)),
            pl.BlockSpec((1, DE), lambda i: (0, 0)),
            pl.BlockSpec((DE, D), lambda i: (0, 0)),
            pl.BlockSpec((1, D), lambda i: (0, 0)),
        ],
        out_specs=pl.BlockSpec((BLK, D), lambda i: (i, 0)),
        out_shape=jax.ShapeDtypeStruct((N_EDGES, D), jnp.float32),
    )(ea, We, be, Wge, bg)


@functools.partial(
    pl.kernel,
    out_type=jax.ShapeDtypeStruct((2 * N_NODES, D), jnp.float32),
    mesh=plsc.VectorSubcoreMesh(core_axis_name="c", subcore_axis_name="s"),
    scratch_types=[
        pltpu.VMEM((CH,), jnp.int32),
        pltpu.VMEM((CH,), jnp.int32),
        pltpu.VMEM((CH, D), jnp.float32),
        pltpu.VMEM((CH, D), jnp.float32),
        pltpu.VMEM_SHARED((N_NODES, D), jnp.float32),
        pltpu.SemaphoreType.DMA,
        pltpu.SemaphoreType.DMA,
    ],
)
def _sc_scatter(node_hbm, edge_hbm, src_hbm, dst_hbm, out_hbm,
                sidx, didx, nbuf, ebuf, acc, sem_n, sem_e):
    c = lax.axis_index("c")
    s = lax.axis_index("s")
    wid = c * NS + s

    def zbody(e, carry):
        for j in range(8):
            ebuf[e, pl.ds(16 * j, 16)] = jnp.zeros((16,), jnp.float32)
        return carry

    lax.fori_loop(0, CH, zbody, 0)
    row0 = s * RPT
    ntiles16 = jnp.where(s == NS - 1, (N_NODES - 15 * RPT) // 16, RPT // 16)

    def zc(i, carry):
        pltpu.sync_copy(ebuf.at[pl.ds(0, 16)], acc.at[pl.ds(row0 + i * 16, 16)])
        return carry

    lax.fori_loop(0, ntiles16, zc, 0)
    plsc.subcore_barrier()

    ebase = wid * E_W

    def chunk(i, carry):
        base = ebase + i * CH
        pltpu.sync_copy(src_hbm.at[pl.ds(base, CH)], sidx)
        pltpu.sync_copy(dst_hbm.at[pl.ds(base, CH)], didx)
        g = pltpu.async_copy(node_hbm.at[sidx], nbuf, sem_n)
        e2 = pltpu.async_copy(edge_hbm.at[pl.ds(base, CH)], ebuf, sem_e)
        g.wait()
        e2.wait()

        def ebody(e, carry2):
            for j in range(8):
                sl = pl.ds(16 * j, 16)
                v = nbuf[e, sl] + ebuf[e, sl]
                ebuf[e, sl] = jnp.maximum(v, 0.01 * v)
            return carry2

        lax.fori_loop(0, CH, ebody, 0)
        pltpu.sync_copy(ebuf, acc.at[didx], add=True)
        return carry

    lax.fori_loop(0, NCHUNK, chunk, 0)
    plsc.subcore_barrier()

    def oc(i, carry):
        pltpu.sync_copy(acc.at[pl.ds(row0 + i * 16, 16)],
                        out_hbm.at[pl.ds(c * N_NODES + row0 + i * 16, 16)])
        return carry

    lax.fori_loop(0, ntiles16, oc, 0)


def _final_add(p):
    def body(a_ref, b_ref, o_ref):
        o_ref[...] = a_ref[...] + b_ref[...]

    return pl.pallas_call(
        body,
        out_shape=jax.ShapeDtypeStruct((N_NODES, D), jnp.float32),
    )(p[:N_NODES], p[N_NODES:])


def kernel(x, edge_index, edge_attr, Wn, bn, We, be, Wg, bg):
    src = edge_index[0].astype(jnp.int32)
    dst = edge_index[1].astype(jnp.int32)
    Wge = Wg[:DE, :]
    Wgn = Wg[DE:, :]
    node_part = _node_dense(x, Wn, bn.reshape(1, D), Wgn)
    edge_part = _edge_dense(edge_attr, We, be.reshape(1, DE), Wge, bg.reshape(1, D))
    partials = _sc_scatter(node_part, edge_part, src, dst)
    return _final_add(partials)


# speedup vs baseline: 4.2508x; 1.5742x over previous
import functools

import jax
import jax.numpy as jnp
from jax import lax
from jax.experimental import pallas as pl
from jax.experimental.pallas import tpu as pltpu
from jax.experimental.pallas import tpu_sc as plsc

N_NODES = 10000
N_EDGES = 320000
D = 128
DE = 16

NC = 2
NS = 16
NW = NC * NS
E_W = N_EDGES // NW
CH = 40
NCHUNK = E_W // CH
RPT = 624


def _leaky(v):
    return jnp.maximum(v, 0.01 * v)


def _node_dense(x, Wn, bn, Wgn):
    def body(x_ref, wn_ref, bn_ref, wgn_ref, o_ref):
        h = jnp.dot(x_ref[...], wn_ref[...], preferred_element_type=jnp.float32)
        h = _leaky(h + bn_ref[...])
        o_ref[...] = jnp.dot(h, wgn_ref[...], preferred_element_type=jnp.float32)

    return pl.pallas_call(
        body,
        out_shape=jax.ShapeDtypeStruct((N_NODES, D), jnp.float32),
    )(x, Wn, bn, Wgn)


def _edge_dense(ea, We, be, Wge, bg):
    BLK = 2560
    grid = N_EDGES // BLK

    def body(a_ref, we_ref, be_ref, wge_ref, bg_ref, o_ref):
        h = _leaky(jnp.dot(a_ref[...], we_ref[...], preferred_element_type=jnp.float32) + be_ref[...])
        o_ref[...] = jnp.dot(h, wge_ref[...], preferred_element_type=jnp.float32) + bg_ref[...]

    return pl.pallas_call(
        body,
        grid=(grid,),
        in_specs=[
            pl.BlockSpec((BLK, DE), lambda i: (i, 0)),
            pl.BlockSpec((DE, DE), lambda i: (0, 0)),
            pl.BlockSpec((1, DE), lambda i: (0, 0)),
            pl.BlockSpec((DE, D), lambda i: (0, 0)),
            pl.BlockSpec((1, D), lambda i: (0, 0)),
        ],
        out_specs=pl.BlockSpec((BLK, D), lambda i: (i, 0)),
        out_shape=jax.ShapeDtypeStruct((N_EDGES, D), jnp.float32),
    )(ea, We, be, Wge, bg)


NBUF = 4


@functools.partial(
    pl.kernel,
    out_type=jax.ShapeDtypeStruct((2 * N_NODES, D), jnp.float32),
    mesh=plsc.VectorSubcoreMesh(core_axis_name="c", subcore_axis_name="s"),
    scratch_types=[
        [pltpu.VMEM((CH,), jnp.int32) for _ in range(NBUF)],
        [pltpu.VMEM((CH,), jnp.int32) for _ in range(NBUF)],
        [pltpu.VMEM((CH, D), jnp.float32) for _ in range(NBUF)],
        [pltpu.VMEM((CH, D), jnp.float32) for _ in range(NBUF)],
        [pltpu.SemaphoreType.DMA for _ in range(NBUF)],
        [pltpu.SemaphoreType.DMA for _ in range(NBUF)],
        [pltpu.SemaphoreType.DMA for _ in range(NBUF)],
        [pltpu.SemaphoreType.DMA for _ in range(NBUF)],
        [pltpu.SemaphoreType.DMA for _ in range(NBUF)],
        pltpu.VMEM_SHARED((N_NODES, D), jnp.float32),
    ],
)
def _sc_scatter(node_hbm, edge_hbm, src_hbm, dst_hbm, out_hbm,
                sidxs, didxs, nbufs, ebufs, sisems, disems, gsems, lsems, ssems, acc):
    c = lax.axis_index("c")
    s = lax.axis_index("s")
    wid = c * NS + s
    ebase = wid * E_W

    z = ebufs[0]

    def zbody(e, carry):
        for j in range(8):
            z[e, pl.ds(16 * j, 16)] = jnp.zeros((16,), jnp.float32)
        return carry

    lax.fori_loop(0, CH, zbody, 0)
    row0 = s * RPT
    ntiles16 = jnp.where(s == NS - 1, (N_NODES - 15 * RPT) // 16, RPT // 16)

    def zc(i, carry):
        pltpu.sync_copy(z.at[pl.ds(0, 16)], acc.at[pl.ds(row0 + i * 16, 16)])
        return carry

    lax.fori_loop(0, ntiles16, zc, 0)
    plsc.subcore_barrier()

    def issue_sidx(i, b):
        pltpu.async_copy(src_hbm.at[pl.ds(ebase + i * CH, CH)], sidxs[b], sisems[b])

    def issue_didx(i, b):
        pltpu.async_copy(dst_hbm.at[pl.ds(ebase + i * CH, CH)], didxs[b], disems[b])

    def issue_data(i, b):
        pltpu.async_copy(node_hbm.at[sidxs[b]], nbufs[b], gsems[b])
        pltpu.async_copy(edge_hbm.at[pl.ds(ebase + i * CH, CH)], ebufs[b], lsems[b])

    def wait_scatter(b):
        pltpu.make_async_copy(ebufs[b], acc.at[didxs[b]], ssems[b]).wait()

    for j in range(2):
        issue_sidx(j, j)
        issue_didx(j, j)
    issue_sidx(2, 2)
    for j in range(2):
        pltpu.make_async_copy(src_hbm.at[pl.ds(0, CH)], sidxs[j], sisems[j]).wait()
        issue_data(j, j)

    def step(i, b):
        b2 = (b + 2) % NBUF
        b3 = (b + 3) % NBUF
        i2 = i + 2
        i3 = i + 3

        @pl.when(i3 < NCHUNK)
        def _():
            issue_sidx(i3, b3)

        @pl.when(i2 < NCHUNK)
        def _():
            @pl.when(i >= 2)
            def _():
                wait_scatter(b2)

            issue_didx(i2, b2)
            pltpu.make_async_copy(src_hbm.at[pl.ds(0, CH)], sidxs[b2], sisems[b2]).wait()
            issue_data(i2, b2)

        pltpu.make_async_copy(node_hbm.at[sidxs[b]], nbufs[b], gsems[b]).wait()
        pltpu.make_async_copy(edge_hbm.at[pl.ds(0, CH)], ebufs[b], lsems[b]).wait()

        nb = nbufs[b]
        eb = ebufs[b]

        def ebody(e, carry2):
            for j in range(8):
                sl = pl.ds(16 * j, 16)
                v = nb[e, sl] + eb[e, sl]
                eb[e, sl] = jnp.maximum(v, 0.01 * v)
            return carry2

        lax.fori_loop(0, CH, ebody, 0)
        pltpu.make_async_copy(dst_hbm.at[pl.ds(0, CH)], didxs[b], disems[b]).wait()
        pltpu.async_copy(eb, acc.at[didxs[b]], ssems[b], add=True)

    def outer(g, carry):
        for db in range(NBUF):
            step(g * NBUF + db, db)
        return carry

    lax.fori_loop(0, NCHUNK // NBUF, outer, 0)
    step(jnp.int32(NCHUNK - 2), (NCHUNK - 2) % NBUF)
    step(jnp.int32(NCHUNK - 1), (NCHUNK - 1) % NBUF)

    for j in range(NBUF):
        wait_scatter(j)

    plsc.subcore_barrier()

    def oc(i, carry):
        pltpu.sync_copy(acc.at[pl.ds(row0 + i * 16, 16)],
                        out_hbm.at[pl.ds(c * N_NODES + row0 + i * 16, 16)])
        return carry

    lax.fori_loop(0, ntiles16, oc, 0)


def _final_add(p):
    def body(a_ref, b_ref, o_ref):
        o_ref[...] = a_ref[...] + b_ref[...]

    return pl.pallas_call(
        body,
        out_shape=jax.ShapeDtypeStruct((N_NODES, D), jnp.float32),
    )(p[:N_NODES], p[N_NODES:])


def kernel(x, edge_index, edge_attr, Wn, bn, We, be, Wg, bg):
    src = edge_index[0].astype(jnp.int32)
    dst = edge_index[1].astype(jnp.int32)
    Wge = Wg[:DE, :]
    Wgn = Wg[DE:, :]
    node_part = _node_dense(x, Wn, bn.reshape(1, D), Wgn)
    edge_part = _edge_dense(edge_attr, We, be.reshape(1, DE), Wge, bg.reshape(1, D))
    partials = _sc_scatter(node_part, edge_part, src, dst)
    return _final_add(partials)


# speedup vs baseline: 6.7851x; 1.5962x over previous
import functools

import jax
import jax.numpy as jnp
from jax import lax
from jax.experimental import pallas as pl
from jax.experimental.pallas import tpu as pltpu
from jax.experimental.pallas import tpu_sc as plsc

N_NODES = 10000
N_EDGES = 320000
D = 128
DE = 16

NC = 2
NS = 16
NW = NC * NS
E_W = N_EDGES // NW
CH = 40
NCHUNK = E_W // CH
RPT = 624


def _leaky(v):
    return jnp.maximum(v, 0.01 * v)


def _node_dense(x, Wn, bn, Wgn):
    def body(x_ref, wn_ref, bn_ref, wgn_ref, o_ref):
        h = jnp.dot(x_ref[...], wn_ref[...], preferred_element_type=jnp.float32)
        h = _leaky(h + bn_ref[...])
        o_ref[...] = jnp.dot(h, wgn_ref[...], preferred_element_type=jnp.float32)

    return pl.pallas_call(
        body,
        out_shape=jax.ShapeDtypeStruct((N_NODES, D), jnp.float32),
    )(x, Wn, bn, Wgn)


def _edge_dense(ea_t, We, be, Wge, bg):
    BLK = 12800
    grid = N_EDGES // BLK
    cdim = (((0,), (0,)), ((), ()))

    def body(a_ref, we_ref, be_ref, wge_ref, bg_ref, o_ref):
        ht = lax.dot_general(we_ref[...], a_ref[...], cdim,
                             preferred_element_type=jnp.float32)
        ht = _leaky(ht + be_ref[...])
        o_ref[...] = lax.dot_general(ht, wge_ref[...], cdim,
                                     preferred_element_type=jnp.float32) + bg_ref[...]

    return pl.pallas_call(
        body,
        grid=(grid,),
        in_specs=[
            pl.BlockSpec((DE, BLK), lambda i: (0, i)),
            pl.BlockSpec((DE, DE), lambda i: (0, 0)),
            pl.BlockSpec((DE, 1), lambda i: (0, 0)),
            pl.BlockSpec((DE, D), lambda i: (0, 0)),
            pl.BlockSpec((1, D), lambda i: (0, 0)),
        ],
        out_specs=pl.BlockSpec((BLK, D), lambda i: (i, 0)),
        out_shape=jax.ShapeDtypeStruct((N_EDGES, D), jnp.float32),
    )(ea_t, We, be, Wge, bg)


NBUF = 4


@functools.partial(
    pl.kernel,
    out_type=jax.ShapeDtypeStruct((2 * N_NODES, D), jnp.float32),
    mesh=plsc.VectorSubcoreMesh(core_axis_name="c", subcore_axis_name="s"),
    scratch_types=[
        [pltpu.VMEM((CH,), jnp.int32) for _ in range(NBUF)],
        [pltpu.VMEM((CH,), jnp.int32) for _ in range(NBUF)],
        [pltpu.VMEM((CH, D), jnp.float32) for _ in range(NBUF)],
        [pltpu.VMEM((CH, D), jnp.float32) for _ in range(NBUF)],
        [pltpu.SemaphoreType.DMA for _ in range(NBUF)],
        [pltpu.SemaphoreType.DMA for _ in range(NBUF)],
        [pltpu.SemaphoreType.DMA for _ in range(NBUF)],
        [pltpu.SemaphoreType.DMA for _ in range(NBUF)],
        [pltpu.SemaphoreType.DMA for _ in range(NBUF)],
        pltpu.VMEM_SHARED((N_NODES, D), jnp.float32),
    ],
)
def _sc_scatter(node_hbm, edge_hbm, src_hbm, dst_hbm, out_hbm,
                sidxs, didxs, nbufs, ebufs, sisems, disems, gsems, lsems, ssems, acc):
    c = lax.axis_index("c")
    s = lax.axis_index("s")
    wid = c * NS + s
    ebase = wid * E_W

    z = ebufs[0]

    def zbody(e, carry):
        for j in range(8):
            z[e, pl.ds(16 * j, 16)] = jnp.zeros((16,), jnp.float32)
        return carry

    lax.fori_loop(0, CH, zbody, 0)
    row0 = s * RPT
    ntiles16 = jnp.where(s == NS - 1, (N_NODES - 15 * RPT) // 16, RPT // 16)

    def zc(i, carry):
        pltpu.sync_copy(z.at[pl.ds(0, 16)], acc.at[pl.ds(row0 + i * 16, 16)])
        return carry

    lax.fori_loop(0, ntiles16, zc, 0)
    plsc.subcore_barrier()

    def issue_sidx(i, b):
        pltpu.async_copy(src_hbm.at[pl.ds(ebase + i * CH, CH)], sidxs[b], sisems[b])

    def issue_didx(i, b):
        pltpu.async_copy(dst_hbm.at[pl.ds(ebase + i * CH, CH)], didxs[b], disems[b])

    def issue_data(i, b):
        pltpu.async_copy(node_hbm.at[sidxs[b]], nbufs[b], gsems[b])
        pltpu.async_copy(edge_hbm.at[pl.ds(ebase + i * CH, CH)], ebufs[b], lsems[b])

    def wait_scatter(b):
        pltpu.make_async_copy(ebufs[b], acc.at[didxs[b]], ssems[b]).wait()

    for j in range(2):
        issue_sidx(j, j)
        issue_didx(j, j)
    issue_sidx(2, 2)
    for j in range(2):
        pltpu.make_async_copy(src_hbm.at[pl.ds(0, CH)], sidxs[j], sisems[j]).wait()
        issue_data(j, j)

    def step(i, b):
        b2 = (b + 2) % NBUF
        b3 = (b + 3) % NBUF
        i2 = i + 2
        i3 = i + 3

        @pl.when(i3 < NCHUNK)
        def _():
            issue_sidx(i3, b3)

        @pl.when(i2 < NCHUNK)
        def _():
            @pl.when(i >= 2)
            def _():
                wait_scatter(b2)

            issue_didx(i2, b2)
            pltpu.make_async_copy(src_hbm.at[pl.ds(0, CH)], sidxs[b2], sisems[b2]).wait()
            issue_data(i2, b2)

        pltpu.make_async_copy(node_hbm.at[sidxs[b]], nbufs[b], gsems[b]).wait()
        pltpu.make_async_copy(edge_hbm.at[pl.ds(0, CH)], ebufs[b], lsems[b]).wait()

        nb = nbufs[b]
        eb = ebufs[b]

        def ebody(e, carry2):
            for j in range(8):
                sl = pl.ds(16 * j, 16)
                v = nb[e, sl] + eb[e, sl]
                eb[e, sl] = jnp.maximum(v, 0.01 * v)
            return carry2

        lax.fori_loop(0, CH, ebody, 0)
        pltpu.make_async_copy(dst_hbm.at[pl.ds(0, CH)], didxs[b], disems[b]).wait()
        pltpu.async_copy(eb, acc.at[didxs[b]], ssems[b], add=True)

    def outer(g, carry):
        for db in range(NBUF):
            step(g * NBUF + db, db)
        return carry

    lax.fori_loop(0, NCHUNK // NBUF, outer, 0)
    step(jnp.int32(NCHUNK - 2), (NCHUNK - 2) % NBUF)
    step(jnp.int32(NCHUNK - 1), (NCHUNK - 1) % NBUF)

    for j in range(NBUF):
        wait_scatter(j)

    plsc.subcore_barrier()

    def oc(i, carry):
        pltpu.sync_copy(acc.at[pl.ds(row0 + i * 16, 16)],
                        out_hbm.at[pl.ds(c * N_NODES + row0 + i * 16, 16)])
        return carry

    lax.fori_loop(0, ntiles16, oc, 0)


def _final_add(p):
    def body(a_ref, b_ref, o_ref):
        o_ref[...] = a_ref[...] + b_ref[...]

    return pl.pallas_call(
        body,
        out_shape=jax.ShapeDtypeStruct((N_NODES, D), jnp.float32),
    )(p[:N_NODES], p[N_NODES:])


def kernel(x, edge_index, edge_attr, Wn, bn, We, be, Wg, bg):
    src = edge_index[0].astype(jnp.int32)
    dst = edge_index[1].astype(jnp.int32)
    Wge = Wg[:DE, :]
    Wgn = Wg[DE:, :]
    node_part = _node_dense(x, Wn, bn.reshape(1, D), Wgn)
    edge_part = _edge_dense(edge_attr.T, We, be.reshape(DE, 1), Wge, bg.reshape(1, D))
    partials = _sc_scatter(node_part, edge_part, src, dst)
    return _final_add(partials)
